# baseline (speedup 1.0000x reference)
import functools
import math

import jax
import jax.numpy as jnp
from jax import lax
from jax.experimental import pallas as pl
from jax.experimental.pallas import tpu as pltpu
from jax.experimental.pallas import tpu_sc as plsc

S = 200
B = 1024
D = 64
SCALE = 8.0
N = 1000000

NC = 2
NS = 16
NW = NC * NS
BGRP = 4
SGRP = NW // BGRP
NPHASE = 4
SQ = S // NPHASE

K0_COLS = 16384


def _make_pe(d_model, max_len):
    position = jnp.arange(0, max_len, dtype=jnp.float32)[:, None]
    div_term = jnp.exp(
        jnp.arange(0, d_model, 2, dtype=jnp.float32) * (-math.log(10000.0) / d_model)
    )
    pe = jnp.zeros((max_len, d_model), dtype=jnp.float32)
    pe = pe.at[:, 0::2].set(jnp.sin(position * div_term))
    pe = pe.at[:, 1::2].set(jnp.cos(position * div_term))
    return pe


def _widen_block(a_ref, o_ref):
    half = K0_COLS // 2
    zeros = jnp.zeros((half, D), jnp.float32)
    t0 = jnp.transpose(a_ref[:, pl.ds(0, half)], (1, 0))
    t1 = jnp.transpose(a_ref[:, pl.ds(half, half)], (1, 0))
    o_ref[pl.ds(0, half), :] = jnp.concatenate([t0, zeros], axis=1)
    o_ref[pl.ds(half, half), :] = jnp.concatenate([t1, zeros], axis=1)


def _widen_table(tab_t):
    grid = (N + K0_COLS - 1) // K0_COLS
    return pl.pallas_call(
        _widen_block,
        grid=(grid,),
        in_specs=[pl.BlockSpec((D, K0_COLS), lambda k: (0, k))],
        out_specs=pl.BlockSpec((K0_COLS, 2 * D), lambda k: (k, 0)),
        out_shape=jax.ShapeDtypeStruct((N, 2 * D), jnp.float32),
    )(tab_t)


def _make_gather(s_count):
    sgrp_n, bgrp_n = 2, 16
    s_per = s_count // sgrp_n
    b_per = B // bgrp_n

    @functools.partial(
        pl.kernel,
        mesh=plsc.VectorSubcoreMesh(core_axis_name="c", subcore_axis_name="s"),
        compiler_params=pltpu.CompilerParams(
            use_tc_tiling_on_sc=False, needs_layout_passes=False
        ),
        out_type=jax.ShapeDtypeStruct((s_count, B, 2 * D), jnp.float32),
        scratch_types=[
            pltpu.VMEM((s_per, 1, 64), jnp.int32),
            pltpu.VMEM((b_per, 2 * D), jnp.float32),
            pltpu.VMEM((b_per, 2 * D), jnp.float32),
            pltpu.SemaphoreType.DMA,
            pltpu.SemaphoreType.DMA,
            pltpu.SemaphoreType.DMA,
            pltpu.SemaphoreType.DMA,
        ],
    )
    def _gather_kernel(xt_hbm, tab_hbm, out_hbm, idx_v, g0, g1, gsem0, gsem1,
                       wsem0, wsem1):
        wid = lax.axis_index("s") * NC + lax.axis_index("c")
        sgrp = wid // bgrp_n
        bq = wid % bgrp_n
        s_lo = sgrp * s_per
        b0 = bq * b_per

        pltpu.sync_copy(xt_hbm.at[pl.ds(s_lo, s_per), pl.ds(bq, 1)], idx_v)

        bufs = (g0, g1)
        gsems = (gsem0, gsem1)
        wsems = (wsem0, wsem1)

        def start_gather(i, buf, gsem):
            return pltpu.async_copy(tab_hbm.at[idx_v.at[i, 0]], buf, gsem)

        pending_g = start_gather(0, bufs[0], gsems[0])
        pending_w = [None, None]
        for i in range(s_per):
            cur = i % 2
            nxt = 1 - cur
            pending_g.wait()
            if i + 1 < s_per:
                if pending_w[nxt] is not None:
                    pending_w[nxt].wait()
                pending_g = start_gather(i + 1, bufs[nxt], gsems[nxt])
            pending_w[cur] = pltpu.async_copy(
                bufs[cur], out_hbm.at[s_lo + i, pl.ds(b0, b_per)], wsems[cur]
            )
        for w in pending_w:
            if w is not None:
                w.wait()

    return _gather_kernel


SB = 2


def _finish_block(a_ref, pe_ref, o_ref):
    for u in range(SB):
        t = jnp.transpose(a_ref[u, :, pl.ds(0, D)], (1, 0))
        pe_col = jnp.expand_dims(pe_ref[u, 0, pl.ds(0, D)], 1)
        o_ref[u] = t * SCALE + pe_col


def _finish_first(gathered, pe_t):
    return pl.pallas_call(
        _finish_block,
        grid=(SQ // SB,),
        in_specs=[
            pl.BlockSpec((SB, B, 2 * D), lambda s: (s, 0, 0)),
            pl.BlockSpec((SB, 1, 2 * D), lambda s: (s, 0, 0)),
        ],
        out_specs=pl.BlockSpec((SB, D, B), lambda s: (s, 0, 0)),
        out_shape=jax.ShapeDtypeStruct((S, D, B), jnp.float32),
    )(gathered, pe_t)


def _finish_next_block(a_ref, pe_ref, prev_ref, o_ref):
    _finish_block(a_ref, pe_ref, o_ref)


def _finish_next(gathered, pe_t, prev, phase):
    base = phase * SQ // SB
    return pl.pallas_call(
        _finish_next_block,
        grid=(SQ // SB,),
        in_specs=[
            pl.BlockSpec((SB, B, 2 * D), lambda s: (s, 0, 0)),
            pl.BlockSpec((SB, 1, 2 * D), lambda s: (s + base, 0, 0)),
            pl.BlockSpec(memory_space=pl.ANY),
        ],
        out_specs=pl.BlockSpec((SB, D, B), lambda s: (s + base, 0, 0)),
        out_shape=jax.ShapeDtypeStruct((S, D, B), jnp.float32),
        input_output_aliases={2: 0},
    )(gathered, pe_t, prev)


def kernel(x, emb_table):
    xt = jnp.transpose(x.astype(jnp.int32), (1, 0))
    xt3 = jnp.reshape(xt, (S, 16, 64))
    tab_t = jnp.transpose(emb_table, (1, 0))
    tab_wide = _widen_table(tab_t)
    pe = _make_pe(D, S)
    pe_t = jnp.concatenate([pe, pe], axis=1).reshape(S, 1, 2 * D)

    gather_q = _make_gather(SQ)
    slabs = [
        gather_q(xt3[p * SQ:(p + 1) * SQ], tab_wide) for p in range(NPHASE)
    ]
    out = _finish_first(slabs[0], pe_t)
    for p in range(1, NPHASE):
        out = _finish_next(slabs[p], pe_t, out, p)
    return jnp.transpose(out, (0, 2, 1))


# speedup vs baseline: 1.0868x; 1.0868x over previous
import functools
import math

import jax
import jax.numpy as jnp
from jax import lax
from jax.experimental import pallas as pl
from jax.experimental.pallas import tpu as pltpu
from jax.experimental.pallas import tpu_sc as plsc

S = 200
B = 1024
D = 64
SCALE = 8.0
N = 1000000

NC = 2
NS = 16
NW = NC * NS
BGRP = 4
SGRP = NW // BGRP
NPHASE = 2
SQ = S // NPHASE

K0_COLS = 16384


def _make_pe(d_model, max_len):
    position = jnp.arange(0, max_len, dtype=jnp.float32)[:, None]
    div_term = jnp.exp(
        jnp.arange(0, d_model, 2, dtype=jnp.float32) * (-math.log(10000.0) / d_model)
    )
    pe = jnp.zeros((max_len, d_model), dtype=jnp.float32)
    pe = pe.at[:, 0::2].set(jnp.sin(position * div_term))
    pe = pe.at[:, 1::2].set(jnp.cos(position * div_term))
    return pe


def _widen_block(a_ref, o_ref):
    half = K0_COLS // 2
    zeros = jnp.zeros((half, D), jnp.float32)
    t0 = jnp.transpose(a_ref[:, pl.ds(0, half)], (1, 0))
    t1 = jnp.transpose(a_ref[:, pl.ds(half, half)], (1, 0))
    o_ref[pl.ds(0, half), :] = jnp.concatenate([t0, zeros], axis=1)
    o_ref[pl.ds(half, half), :] = jnp.concatenate([t1, zeros], axis=1)


def _widen_table(tab_t):
    grid = (N + K0_COLS - 1) // K0_COLS
    return pl.pallas_call(
        _widen_block,
        grid=(grid,),
        in_specs=[pl.BlockSpec((D, K0_COLS), lambda k: (0, k))],
        out_specs=pl.BlockSpec((K0_COLS, 2 * D), lambda k: (k, 0)),
        out_shape=jax.ShapeDtypeStruct((N, 2 * D), jnp.float32),
    )(tab_t)


def _make_gather(s_count):
    sgrp_n, bgrp_n = 4, 8
    s_per = s_count // sgrp_n
    b_per = B // bgrp_n

    @functools.partial(
        pl.kernel,
        mesh=plsc.VectorSubcoreMesh(core_axis_name="c", subcore_axis_name="s"),
        compiler_params=pltpu.CompilerParams(
            use_tc_tiling_on_sc=False, needs_layout_passes=False
        ),
        out_type=jax.ShapeDtypeStruct((s_count, B, 2 * D), jnp.float32),
        scratch_types=[
            pltpu.VMEM((s_per, 1, 128), jnp.int32),
            pltpu.VMEM((b_per, 2 * D), jnp.float32),
            pltpu.VMEM((b_per, 2 * D), jnp.float32),
            pltpu.SemaphoreType.DMA,
            pltpu.SemaphoreType.DMA,
            pltpu.SemaphoreType.DMA,
            pltpu.SemaphoreType.DMA,
        ],
    )
    def _gather_kernel(xt_hbm, tab_hbm, out_hbm, idx_v, g0, g1, gsem0, gsem1,
                       wsem0, wsem1):
        wid = lax.axis_index("s") * NC + lax.axis_index("c")
        sgrp = wid // bgrp_n
        bq = wid % bgrp_n
        s_lo = sgrp * s_per
        b0 = bq * b_per

        pltpu.sync_copy(xt_hbm.at[pl.ds(s_lo, s_per), pl.ds(bq, 1)], idx_v)

        bufs = (g0, g1)
        gsems = (gsem0, gsem1)
        wsems = (wsem0, wsem1)

        def start_gather(i, buf, gsem):
            return pltpu.async_copy(tab_hbm.at[idx_v.at[i, 0]], buf, gsem)

        pending_g = start_gather(0, bufs[0], gsems[0])
        pending_w = [None, None]
        for i in range(s_per):
            cur = i % 2
            nxt = 1 - cur
            pending_g.wait()
            if i + 1 < s_per:
                if pending_w[nxt] is not None:
                    pending_w[nxt].wait()
                pending_g = start_gather(i + 1, bufs[nxt], gsems[nxt])
            pending_w[cur] = pltpu.async_copy(
                bufs[cur], out_hbm.at[s_lo + i, pl.ds(b0, b_per)], wsems[cur]
            )
        for w in pending_w:
            if w is not None:
                w.wait()

    return _gather_kernel


SB = 4


def _finish_block(a_ref, pe_ref, o_ref):
    for u in range(SB):
        t = jnp.transpose(a_ref[u, :, pl.ds(0, D)], (1, 0))
        pe_col = jnp.expand_dims(pe_ref[u, 0, pl.ds(0, D)], 1)
        o_ref[u] = t * SCALE + pe_col


def _finish_first(gathered, pe_t):
    return pl.pallas_call(
        _finish_block,
        grid=(SQ // SB,),
        in_specs=[
            pl.BlockSpec((SB, B, 2 * D), lambda s: (s, 0, 0)),
            pl.BlockSpec((SB, 1, 2 * D), lambda s: (s, 0, 0)),
        ],
        out_specs=pl.BlockSpec((SB, D, B), lambda s: (s, 0, 0)),
        out_shape=jax.ShapeDtypeStruct((S, D, B), jnp.float32),
    )(gathered, pe_t)


def _finish_next_block(a_ref, pe_ref, prev_ref, o_ref):
    _finish_block(a_ref, pe_ref, o_ref)


def _finish_next(gathered, pe_t, prev, phase):
    base = phase * SQ // SB
    return pl.pallas_call(
        _finish_next_block,
        grid=(SQ // SB,),
        in_specs=[
            pl.BlockSpec((SB, B, 2 * D), lambda s: (s, 0, 0)),
            pl.BlockSpec((SB, 1, 2 * D), lambda s: (s + base, 0, 0)),
            pl.BlockSpec(memory_space=pl.ANY),
        ],
        out_specs=pl.BlockSpec((SB, D, B), lambda s: (s + base, 0, 0)),
        out_shape=jax.ShapeDtypeStruct((S, D, B), jnp.float32),
        input_output_aliases={2: 0},
    )(gathered, pe_t, prev)


def kernel(x, emb_table):
    xt = jnp.transpose(x.astype(jnp.int32), (1, 0))
    xt3 = jnp.reshape(xt, (S, 8, 128))
    tab_t = jnp.transpose(emb_table, (1, 0))
    tab_wide = _widen_table(tab_t)
    pe = _make_pe(D, S)
    pe_t = jnp.concatenate([pe, pe], axis=1).reshape(S, 1, 2 * D)

    gather_q = _make_gather(SQ)
    slabs = [
        gather_q(xt3[p * SQ:(p + 1) * SQ], tab_wide) for p in range(NPHASE)
    ]
    out = _finish_first(slabs[0], pe_t)
    for p in range(1, NPHASE):
        out = _finish_next(slabs[p], pe_t, out, p)
    return jnp.transpose(out, (0, 2, 1))


# speedup vs baseline: 1.1282x; 1.0380x over previous
import functools
import math

import jax
import jax.numpy as jnp
from jax import lax
from jax.experimental import pallas as pl
from jax.experimental.pallas import tpu as pltpu
from jax.experimental.pallas import tpu_sc as plsc

S = 200
B = 1024
D = 64
SCALE = 8.0
N = 1000000

NC = 2
NS = 16
NW = NC * NS
BGRP = 4
SGRP = NW // BGRP
NPHASE = 2
SQ = S // NPHASE

K0_COLS = 32768


def _make_pe(d_model, max_len):
    position = jnp.arange(0, max_len, dtype=jnp.float32)[:, None]
    div_term = jnp.exp(
        jnp.arange(0, d_model, 2, dtype=jnp.float32) * (-math.log(10000.0) / d_model)
    )
    pe = jnp.zeros((max_len, d_model), dtype=jnp.float32)
    pe = pe.at[:, 0::2].set(jnp.sin(position * div_term))
    pe = pe.at[:, 1::2].set(jnp.cos(position * div_term))
    return pe


def _widen_block(a_ref, o_ref):
    half = K0_COLS // 2
    zeros = jnp.zeros((half, D), jnp.float32)
    t0 = jnp.transpose(a_ref[:, pl.ds(0, half)], (1, 0))
    t1 = jnp.transpose(a_ref[:, pl.ds(half, half)], (1, 0))
    o_ref[pl.ds(0, half), :] = jnp.concatenate([t0, zeros], axis=1)
    o_ref[pl.ds(half, half), :] = jnp.concatenate([t1, zeros], axis=1)


def _widen_table(tab_t):
    grid = (N + K0_COLS - 1) // K0_COLS
    return pl.pallas_call(
        _widen_block,
        grid=(grid,),
        in_specs=[pl.BlockSpec((D, K0_COLS), lambda k: (0, k))],
        out_specs=pl.BlockSpec((K0_COLS, 2 * D), lambda k: (k, 0)),
        out_shape=jax.ShapeDtypeStruct((N, 2 * D), jnp.float32),
    )(tab_t)


def _make_gather(s_count):
    sgrp_n, bgrp_n = 4, 8
    s_per = s_count // sgrp_n
    b_per = B // bgrp_n

    @functools.partial(
        pl.kernel,
        mesh=plsc.VectorSubcoreMesh(core_axis_name="c", subcore_axis_name="s"),
        compiler_params=pltpu.CompilerParams(
            use_tc_tiling_on_sc=False, needs_layout_passes=False
        ),
        out_type=jax.ShapeDtypeStruct((s_count, B, 2 * D), jnp.float32),
        scratch_types=[
            pltpu.VMEM((s_per, 1, 128), jnp.int32),
            pltpu.VMEM((b_per, 2 * D), jnp.float32),
            pltpu.VMEM((b_per, 2 * D), jnp.float32),
            pltpu.SemaphoreType.DMA,
            pltpu.SemaphoreType.DMA,
            pltpu.SemaphoreType.DMA,
            pltpu.SemaphoreType.DMA,
        ],
    )
    def _gather_kernel(xt_hbm, tab_hbm, out_hbm, idx_v, g0, g1, gsem0, gsem1,
                       wsem0, wsem1):
        wid = lax.axis_index("s") * NC + lax.axis_index("c")
        sgrp = wid // bgrp_n
        bq = wid % bgrp_n
        s_lo = sgrp * s_per
        b0 = bq * b_per

        pltpu.sync_copy(xt_hbm.at[pl.ds(s_lo, s_per), pl.ds(bq, 1)], idx_v)

        bufs = (g0, g1)
        gsems = (gsem0, gsem1)
        wsems = (wsem0, wsem1)

        def start_gather(i, buf, gsem):
            return pltpu.async_copy(tab_hbm.at[idx_v.at[i, 0]], buf, gsem)

        pending_g = start_gather(0, bufs[0], gsems[0])
        pending_w = [None, None]
        for i in range(s_per):
            cur = i % 2
            nxt = 1 - cur
            pending_g.wait()
            if i + 1 < s_per:
                if pending_w[nxt] is not None:
                    pending_w[nxt].wait()
                pending_g = start_gather(i + 1, bufs[nxt], gsems[nxt])
            pending_w[cur] = pltpu.async_copy(
                bufs[cur], out_hbm.at[s_lo + i, pl.ds(b0, b_per)], wsems[cur]
            )
        for w in pending_w:
            if w is not None:
                w.wait()

    return _gather_kernel


SB = 10


def _finish_block(a_ref, pe_ref, o_ref):
    for u in range(SB):
        t = jnp.transpose(a_ref[u, :, pl.ds(0, D)], (1, 0))
        pe_col = jnp.expand_dims(pe_ref[u, 0, pl.ds(0, D)], 1)
        o_ref[u] = t * SCALE + pe_col


def _finish_first(gathered, pe_t):
    return pl.pallas_call(
        _finish_block,
        grid=(SQ // SB,),
        in_specs=[
            pl.BlockSpec((SB, B, 2 * D), lambda s: (s, 0, 0)),
            pl.BlockSpec((SB, 1, 2 * D), lambda s: (s, 0, 0)),
        ],
        out_specs=pl.BlockSpec((SB, D, B), lambda s: (s, 0, 0)),
        out_shape=jax.ShapeDtypeStruct((S, D, B), jnp.float32),
    )(gathered, pe_t)


def _finish_next_block(a_ref, pe_ref, prev_ref, o_ref):
    _finish_block(a_ref, pe_ref, o_ref)


def _finish_next(gathered, pe_t, prev, phase):
    base = phase * SQ // SB
    return pl.pallas_call(
        _finish_next_block,
        grid=(SQ // SB,),
        in_specs=[
            pl.BlockSpec((SB, B, 2 * D), lambda s: (s, 0, 0)),
            pl.BlockSpec((SB, 1, 2 * D), lambda s: (s + base, 0, 0)),
            pl.BlockSpec(memory_space=pl.ANY),
        ],
        out_specs=pl.BlockSpec((SB, D, B), lambda s: (s + base, 0, 0)),
        out_shape=jax.ShapeDtypeStruct((S, D, B), jnp.float32),
        input_output_aliases={2: 0},
    )(gathered, pe_t, prev)


def kernel(x, emb_table):
    xt = jnp.transpose(x.astype(jnp.int32), (1, 0))
    xt3 = jnp.reshape(xt, (S, 8, 128))
    tab_t = jnp.transpose(emb_table, (1, 0))
    tab_wide = _widen_table(tab_t)
    pe = _make_pe(D, S)
    pe_t = jnp.concatenate([pe, pe], axis=1).reshape(S, 1, 2 * D)

    gather_q = _make_gather(SQ)
    slabs = [
        gather_q(xt3[p * SQ:(p + 1) * SQ], tab_wide) for p in range(NPHASE)
    ]
    out = _finish_first(slabs[0], pe_t)
    for p in range(1, NPHASE):
        out = _finish_next(slabs[p], pe_t, out, p)
    return jnp.transpose(out, (0, 2, 1))


# speedup vs baseline: 1.1300x; 1.0016x over previous
import functools
import math

import jax
import jax.numpy as jnp
from jax import lax
from jax.experimental import pallas as pl
from jax.experimental.pallas import tpu as pltpu
from jax.experimental.pallas import tpu_sc as plsc

S = 200
B = 1024
D = 64
SCALE = 8.0
N = 1000000

NC = 2
NS = 16
NW = NC * NS
BGRP = 4
SGRP = NW // BGRP
NPHASE = 2
SQ = S // NPHASE

K0_COLS = 32768


def _make_pe(d_model, max_len):
    position = jnp.arange(0, max_len, dtype=jnp.float32)[:, None]
    div_term = jnp.exp(
        jnp.arange(0, d_model, 2, dtype=jnp.float32) * (-math.log(10000.0) / d_model)
    )
    pe = jnp.zeros((max_len, d_model), dtype=jnp.float32)
    pe = pe.at[:, 0::2].set(jnp.sin(position * div_term))
    pe = pe.at[:, 1::2].set(jnp.cos(position * div_term))
    return pe


def _widen_block(a_ref, o_ref):
    half = K0_COLS // 2
    zeros = jnp.zeros((half, D), jnp.float32)
    t0 = jnp.transpose(a_ref[:, pl.ds(0, half)], (1, 0))
    t1 = jnp.transpose(a_ref[:, pl.ds(half, half)], (1, 0))
    o_ref[pl.ds(0, half), :] = jnp.concatenate([t0, zeros], axis=1)
    o_ref[pl.ds(half, half), :] = jnp.concatenate([t1, zeros], axis=1)


def _widen_table(tab_t):
    grid = (N + K0_COLS - 1) // K0_COLS
    return pl.pallas_call(
        _widen_block,
        grid=(grid,),
        in_specs=[pl.BlockSpec((D, K0_COLS), lambda k: (0, k))],
        out_specs=pl.BlockSpec((K0_COLS, 2 * D), lambda k: (k, 0)),
        out_shape=jax.ShapeDtypeStruct((N, 2 * D), jnp.float32),
    )(tab_t)


def _make_gather(s_count):
    sgrp_n, bgrp_n = 4, 8
    s_per = s_count // sgrp_n
    b_per = B // bgrp_n

    @functools.partial(
        pl.kernel,
        mesh=plsc.VectorSubcoreMesh(core_axis_name="c", subcore_axis_name="s"),
        compiler_params=pltpu.CompilerParams(
            use_tc_tiling_on_sc=False, needs_layout_passes=False
        ),
        out_type=jax.ShapeDtypeStruct((s_count, B, 2 * D), jnp.float32),
        scratch_types=[
            pltpu.VMEM((s_per, 1, 128), jnp.int32),
            pltpu.VMEM((b_per, 2 * D), jnp.float32),
            pltpu.VMEM((b_per, 2 * D), jnp.float32),
            pltpu.SemaphoreType.DMA,
            pltpu.SemaphoreType.DMA,
            pltpu.SemaphoreType.DMA,
            pltpu.SemaphoreType.DMA,
        ],
    )
    def _gather_kernel(xt_hbm, tab_hbm, out_hbm, idx_v, g0, g1, gsem0, gsem1,
                       wsem0, wsem1):
        wid = lax.axis_index("s") * NC + lax.axis_index("c")
        sgrp = wid // bgrp_n
        bq = wid % bgrp_n
        s_lo = sgrp * s_per
        b0 = bq * b_per

        pltpu.sync_copy(xt_hbm.at[pl.ds(s_lo, s_per), pl.ds(bq, 1)], idx_v)

        bufs = (g0, g1)
        gsems = (gsem0, gsem1)
        wsems = (wsem0, wsem1)

        def start_gather(i, buf, gsem):
            return pltpu.async_copy(tab_hbm.at[idx_v.at[i, 0]], buf, gsem)

        pending_g = start_gather(0, bufs[0], gsems[0])
        pending_w = [None, None]
        for i in range(s_per):
            cur = i % 2
            nxt = 1 - cur
            pending_g.wait()
            if i + 1 < s_per:
                if pending_w[nxt] is not None:
                    pending_w[nxt].wait()
                pending_g = start_gather(i + 1, bufs[nxt], gsems[nxt])
            pending_w[cur] = pltpu.async_copy(
                bufs[cur], out_hbm.at[s_lo + i, pl.ds(b0, b_per)], wsems[cur]
            )
        for w in pending_w:
            if w is not None:
                w.wait()

    return _gather_kernel


SB = 20


def _finish_block(a_ref, pe_ref, o_ref):
    for u in range(SB):
        t = jnp.transpose(a_ref[u, :, pl.ds(0, D)], (1, 0))
        pe_col = jnp.expand_dims(pe_ref[u, 0, pl.ds(0, D)], 1)
        o_ref[u] = t * SCALE + pe_col


def _finish_first(gathered, pe_t):
    return pl.pallas_call(
        _finish_block,
        grid=(SQ // SB,),
        in_specs=[
            pl.BlockSpec((SB, B, 2 * D), lambda s: (s, 0, 0)),
            pl.BlockSpec((SB, 1, 2 * D), lambda s: (s, 0, 0)),
        ],
        out_specs=pl.BlockSpec((SB, D, B), lambda s: (s, 0, 0)),
        out_shape=jax.ShapeDtypeStruct((S, D, B), jnp.float32),
    )(gathered, pe_t)


def _finish_next_block(a_ref, pe_ref, prev_ref, o_ref):
    _finish_block(a_ref, pe_ref, o_ref)


def _finish_next(gathered, pe_t, prev, phase):
    base = phase * SQ // SB
    return pl.pallas_call(
        _finish_next_block,
        grid=(SQ // SB,),
        in_specs=[
            pl.BlockSpec((SB, B, 2 * D), lambda s: (s, 0, 0)),
            pl.BlockSpec((SB, 1, 2 * D), lambda s: (s + base, 0, 0)),
            pl.BlockSpec(memory_space=pl.ANY),
        ],
        out_specs=pl.BlockSpec((SB, D, B), lambda s: (s + base, 0, 0)),
        out_shape=jax.ShapeDtypeStruct((S, D, B), jnp.float32),
        input_output_aliases={2: 0},
    )(gathered, pe_t, prev)


def kernel(x, emb_table):
    xt = jnp.transpose(x.astype(jnp.int32), (1, 0))
    xt3 = jnp.reshape(xt, (S, 8, 128))
    tab_t = jnp.transpose(emb_table, (1, 0))
    tab_wide = _widen_table(tab_t)
    pe = _make_pe(D, S)
    pe_t = jnp.concatenate([pe, pe], axis=1).reshape(S, 1, 2 * D)

    gather_q = _make_gather(SQ)
    slabs = [
        gather_q(xt3[p * SQ:(p + 1) * SQ], tab_wide) for p in range(NPHASE)
    ]
    out = _finish_first(slabs[0], pe_t)
    for p in range(1, NPHASE):
        out = _finish_next(slabs[p], pe_t, out, p)
    return jnp.transpose(out, (0, 2, 1))
